# baseline (speedup 1.0000x reference)
import jax
import jax.numpy as jnp
from jax.experimental import pallas as pl

_BLK = 1000


def _mlp_kernel(x_ref, w0_ref, b0_ref, w1_ref, b1_ref, w2_ref, b2_ref, out_ref):
    h = jnp.dot(x_ref[...], w0_ref[...], preferred_element_type=jnp.float32)
    h = h + b0_ref[...]
    h = jnp.dot(h, w1_ref[...], preferred_element_type=jnp.float32)
    h = jnp.maximum(h + b1_ref[...], 0.0)
    out = jnp.dot(h, w2_ref[...], preferred_element_type=jnp.float32)
    out_ref[...] = out + b2_ref[...]


def kernel(x, edge_index, W0, b0, W1, b1, W2, b2):
    del edge_index
    n, d = x.shape
    hid = W0.shape[1]
    end_hid = W1.shape[1]
    out_dim = W2.shape[1]
    grid = n // _BLK
    return pl.pallas_call(
        _mlp_kernel,
        grid=(grid,),
        in_specs=[
            pl.BlockSpec((_BLK, d), lambda i: (i, 0)),
            pl.BlockSpec((d, hid), lambda i: (0, 0)),
            pl.BlockSpec((1, hid), lambda i: (0, 0)),
            pl.BlockSpec((hid, end_hid), lambda i: (0, 0)),
            pl.BlockSpec((1, end_hid), lambda i: (0, 0)),
            pl.BlockSpec((end_hid, out_dim), lambda i: (0, 0)),
            pl.BlockSpec((1, out_dim), lambda i: (0, 0)),
        ],
        out_specs=pl.BlockSpec((_BLK, out_dim), lambda i: (i, 0)),
        out_shape=jax.ShapeDtypeStruct((n, out_dim), jnp.float32),
    )(x, W0, b0.reshape(1, hid), W1, b1.reshape(1, end_hid),
      W2, b2.reshape(1, out_dim))


# speedup vs baseline: 1.5800x; 1.5800x over previous
import jax
import jax.numpy as jnp
from jax import lax
from jax.experimental import pallas as pl
from jax.experimental.pallas import tpu as pltpu

_BLK = 2000


def _mlp_kernel(x_ref, w0_ref, b0_ref, w1_ref, b1_ref, w2_ref, b2_ref, out_ref):
    w01 = jnp.dot(w0_ref[...], w1_ref[...], preferred_element_type=jnp.float32)
    b01_t = lax.dot_general(w1_ref[...], b0_ref[...],
                            (((0,), (1,)), ((), ())),
                            preferred_element_type=jnp.float32) + b1_ref[...]
    h_t = lax.dot_general(w01, x_ref[...], (((0,), (1,)), ((), ())),
                          preferred_element_type=jnp.float32)
    h_t = jnp.maximum(h_t + b01_t, 0.0)
    out_t = lax.dot_general(w2_ref[...], h_t, (((0,), (0,)), ((), ())),
                            preferred_element_type=jnp.float32)
    out_ref[...] = (out_t + b2_ref[...]).reshape(1, 1, _BLK)


def kernel(x, edge_index, W0, b0, W1, b1, W2, b2):
    del edge_index
    n, d = x.shape
    hid = W0.shape[1]
    end_hid = W1.shape[1]
    out_dim = W2.shape[1]
    grid = n // _BLK
    out = pl.pallas_call(
        _mlp_kernel,
        grid=(grid,),
        in_specs=[
            pl.BlockSpec((_BLK, d), lambda i: (i, 0)),
            pl.BlockSpec((d, hid), lambda i: (0, 0)),
            pl.BlockSpec((1, hid), lambda i: (0, 0)),
            pl.BlockSpec((hid, end_hid), lambda i: (0, 0)),
            pl.BlockSpec((end_hid, 1), lambda i: (0, 0)),
            pl.BlockSpec((end_hid, out_dim), lambda i: (0, 0)),
            pl.BlockSpec((1, out_dim), lambda i: (0, 0)),
        ],
        out_specs=pl.BlockSpec((1, 1, _BLK), lambda i: (i, 0, 0)),
        out_shape=jax.ShapeDtypeStruct((grid, 1, _BLK), jnp.float32),
        compiler_params=pltpu.CompilerParams(
            dimension_semantics=("arbitrary",)),
    )(x, W0, b0.reshape(1, hid), W1, b1.reshape(end_hid, 1),
      W2, b2.reshape(1, out_dim))
    return out.reshape(n, out_dim)


# speedup vs baseline: 2.4502x; 1.5508x over previous
import jax
import jax.numpy as jnp
from jax import lax
from jax.experimental import pallas as pl
from jax.experimental.pallas import tpu as pltpu

_BLK = 1000
_WAYS = 5


def _mlp_kernel(x0, x1, x2, x3, x4, w0t_ref, b0_ref, w1_ref, b1_ref, w2r_ref,
                b2_ref, out_ref):
    w1 = w1_ref[...]
    w01t = lax.dot_general(w1, w0t_ref[...], (((0,), (0,)), ((), ())),
                           preferred_element_type=jnp.float32)
    b01t = lax.dot_general(w1, b0_ref[...], (((0,), (1,)), ((), ())),
                           preferred_element_type=jnp.float32) + b1_ref[...].T
    w2r = w2r_ref[...]
    b2 = b2_ref[...]
    outs = []
    for xr in (x0, x1, x2, x3, x4):
        h_t = lax.dot_general(w01t, xr[...], (((1,), (1,)), ((), ())),
                              preferred_element_type=jnp.float32)
        h_t = jnp.maximum(h_t + b01t, 0.0)
        outs.append(lax.dot_general(w2r, h_t, (((1,), (0,)), ((), ())),
                                    preferred_element_type=jnp.float32))
    o = jnp.concatenate(outs, axis=1) + b2
    out_ref[...] = o.reshape(1, 1, _BLK * _WAYS)


def kernel(x, edge_index, W0, b0, W1, b1, W2, b2):
    del edge_index
    n, d = x.shape
    hid = W0.shape[1]
    end_hid = W1.shape[1]
    out_dim = W2.shape[1]
    grid = n // (_BLK * _WAYS)
    x_specs = [
        pl.BlockSpec((_BLK, d), lambda i, j=j: (_WAYS * i + j, 0))
        for j in range(_WAYS)
    ]
    out = pl.pallas_call(
        _mlp_kernel,
        grid=(grid,),
        in_specs=x_specs + [
            pl.BlockSpec((hid, d), lambda i: (0, 0)),
            pl.BlockSpec((1, hid), lambda i: (0, 0)),
            pl.BlockSpec((hid, end_hid), lambda i: (0, 0)),
            pl.BlockSpec((1, end_hid), lambda i: (0, 0)),
            pl.BlockSpec((1, end_hid), lambda i: (0, 0)),
            pl.BlockSpec((1, out_dim), lambda i: (0, 0)),
        ],
        out_specs=pl.BlockSpec((1, 1, _BLK * _WAYS), lambda i: (i, 0, 0)),
        out_shape=jax.ShapeDtypeStruct((grid, 1, _BLK * _WAYS), jnp.float32),
        compiler_params=pltpu.CompilerParams(
            dimension_semantics=("arbitrary",)),
    )(x, x, x, x, x, W0.T, b0.reshape(1, hid), W1, b1.reshape(1, end_hid),
      W2.reshape(1, end_hid), b2.reshape(1, out_dim))
    return out.reshape(n, out_dim)


# speedup vs baseline: 2.8002x; 1.1428x over previous
import jax
import jax.numpy as jnp
from jax import lax
from jax.experimental import pallas as pl
from jax.experimental.pallas import tpu as pltpu

_CHUNK = 1000
_N = 10000
_NCHUNKS = _N // _CHUNK
_NBUF = 5


def _mlp_kernel(x_hbm, w0t_ref, b0_ref, w1_ref, b1_ref, w2r_ref, b2_ref,
                out_ref, buf, sem):
    def copy(c):
        slot = c % _NBUF
        return pltpu.make_async_copy(
            x_hbm.at[pl.ds(c * _CHUNK, _CHUNK), :], buf.at[slot], sem.at[slot])

    for c in range(_NBUF):
        copy(c).start()

    w1 = w1_ref[...]
    w01t = lax.dot_general(w1, w0t_ref[...], (((0,), (0,)), ((), ())),
                           preferred_element_type=jnp.float32)
    b01t = lax.dot_general(w1, b0_ref[...], (((0,), (1,)), ((), ())),
                           preferred_element_type=jnp.float32) + b1_ref[...].T
    w2r = w2r_ref[...]

    outs = []
    for c in range(_NCHUNKS):
        copy(c).wait()
        xb = buf[c % _NBUF]
        if c + _NBUF < _NCHUNKS:
            copy(c + _NBUF).start()
        h_t = lax.dot_general(w01t, xb, (((1,), (1,)), ((), ())),
                              preferred_element_type=jnp.float32)
        h_t = jnp.maximum(h_t + b01t, 0.0)
        outs.append(lax.dot_general(w2r, h_t, (((1,), (0,)), ((), ())),
                                    preferred_element_type=jnp.float32))
    out_ref[...] = jnp.concatenate(outs, axis=1) + b2_ref[...]


def kernel(x, edge_index, W0, b0, W1, b1, W2, b2):
    del edge_index
    n, d = x.shape
    hid = W0.shape[1]
    end_hid = W1.shape[1]
    out_dim = W2.shape[1]
    out = pl.pallas_call(
        _mlp_kernel,
        in_specs=[
            pl.BlockSpec(memory_space=pl.ANY),
            pl.BlockSpec((hid, d), lambda: (0, 0)),
            pl.BlockSpec((1, hid), lambda: (0, 0)),
            pl.BlockSpec((hid, end_hid), lambda: (0, 0)),
            pl.BlockSpec((1, end_hid), lambda: (0, 0)),
            pl.BlockSpec((1, end_hid), lambda: (0, 0)),
            pl.BlockSpec((1, out_dim), lambda: (0, 0)),
        ],
        out_specs=pl.BlockSpec((1, n), lambda: (0, 0)),
        out_shape=jax.ShapeDtypeStruct((1, n), jnp.float32),
        scratch_shapes=[
            pltpu.VMEM((_NBUF, _CHUNK, 128), jnp.float32),
            pltpu.SemaphoreType.DMA((_NBUF,)),
        ],
    )(x, W0.T, b0.reshape(1, hid), W1, b1.reshape(1, end_hid),
      W2.reshape(1, end_hid), b2.reshape(1, out_dim))
    return out.reshape(n, out_dim)
